# baseline (speedup 1.0000x reference)
import functools

import jax
import jax.numpy as jnp
from jax import lax
from jax.experimental import pallas as pl
from jax.experimental.pallas import tpu as pltpu
from jax.experimental.pallas import tpu_sc as plsc

N = 10000
E = 160000
D = 256
K = 27
H = 128

NPAD = 10240
EPAD = 163840
NTILE = 32
EPT = EPAD // NTILE
CHS = 128
NCHS = EPT // CHS
CHA = 32
NCHA = EPT // CHA
RB = 2048
NRB = NPAD // RB


def _mm_body(x_ref, w_ref, b_ref, o_ref):
    o_ref[...] = (
        jnp.dot(x_ref[...], w_ref[...], preferred_element_type=jnp.float32)
        + b_ref[...]
    )


def _conv1(x, W1, b1):
    return pl.pallas_call(
        _mm_body,
        grid=(NRB,),
        in_specs=[
            pl.BlockSpec((RB, D), lambda i: (i, 0)),
            pl.BlockSpec((D, D), lambda i: (0, 0)),
            pl.BlockSpec((1, D), lambda i: (0, 0)),
        ],
        out_specs=pl.BlockSpec((RB, D), lambda i: (i, 0)),
        out_shape=jax.ShapeDtypeStruct((NPAD, D), jnp.float32),
    )(x, W1, b1.reshape(1, D))


def _qkv_body(h_ref, w_ref, b_ref, o_ref):
    tx = h_ref[:, H:]
    o_ref[0] = (
        jnp.dot(tx, w_ref[0], preferred_element_type=jnp.float32) + b_ref[0]
    )


def _qkv(h, Wqkv, bqkv):
    return pl.pallas_call(
        _qkv_body,
        grid=(3, NRB),
        in_specs=[
            pl.BlockSpec((RB, D), lambda p, i: (i, 0)),
            pl.BlockSpec((1, H, H), lambda p, i: (p, 0, 0)),
            pl.BlockSpec((1, 1, H), lambda p, i: (p, 0, 0)),
        ],
        out_specs=pl.BlockSpec((1, RB, H), lambda p, i: (p, i, 0)),
        out_shape=jax.ShapeDtypeStruct((3, NPAD, H), jnp.float32),
    )(h, Wqkv, bqkv)


def _taps_body(f_ref, w_ref, o_ref):
    o_ref[...] = jnp.dot(
        f_ref[...], w_ref[...], preferred_element_type=jnp.float32
    )


def _taps_from_h(h, Wcat):
    def body(h_ref, w_ref, o_ref):
        o_ref[0] = jnp.dot(
            h_ref[:, :H], w_ref[...], preferred_element_type=jnp.float32
        )

    return pl.pallas_call(
        body,
        grid=(K, NRB),
        in_specs=[
            pl.BlockSpec((RB, D), lambda k, i: (i, 0)),
            pl.BlockSpec((H, H), lambda k, i: (0, k)),
        ],
        out_specs=pl.BlockSpec((1, RB, H), lambda k, i: (k, i, 0)),
        out_shape=jax.ShapeDtypeStruct((K, NPAD, H), jnp.float32),
    )(h, Wcat)


def _mid_body(p_ref, b_ref, w_ref, o_ref):
    f2 = jnp.maximum(p_ref[0] + p_ref[1] + b_ref[...], 0.0)
    o_ref[0] = jnp.dot(f2, w_ref[...], preferred_element_type=jnp.float32)


def _mid(p1, br1, Wcat2):
    return pl.pallas_call(
        _mid_body,
        grid=(K, NRB),
        in_specs=[
            pl.BlockSpec((2, RB, H), lambda k, i: (0, i, 0)),
            pl.BlockSpec((1, H), lambda k, i: (0, 0)),
            pl.BlockSpec((H, H), lambda k, i: (0, k)),
        ],
        out_specs=pl.BlockSpec((1, RB, H), lambda k, i: (k, i, 0)),
        out_shape=jax.ShapeDtypeStruct((K, NPAD, H), jnp.float32),
    )(p1, br1.reshape(1, H), Wcat2)


def _final_body(x_ref, h_ref, p2_ref, ag_ref, dn_ref, br2_ref, wo_ref, bo_ref,
                w2a_ref, w2b_ref, b2_ref, o_ref):
    conv_x = h_ref[:, :H]
    trans_x = h_ref[:, H:]
    out2 = jnp.maximum(p2_ref[0] + p2_ref[1] + br2_ref[...], 0.0)
    cx2 = out2 + 2.0 * conv_x
    ag = ag_ref[0] + ag_ref[1]
    denom = jnp.sum(dn_ref[...], axis=0)[:, None] + 1e-9
    agg = ag / denom
    tr = (
        jnp.dot(agg, wo_ref[...], preferred_element_type=jnp.float32)
        + bo_ref[...] + trans_x
    )
    res = (
        jnp.dot(cx2, w2a_ref[...], preferred_element_type=jnp.float32)
        + jnp.dot(tr, w2b_ref[...], preferred_element_type=jnp.float32)
        + b2_ref[...]
    )
    o_ref[...] = x_ref[...] + res


def _final(x, h, p2, aggp, denp, br2, Wo, bo, W2, b2):
    return pl.pallas_call(
        _final_body,
        grid=(NRB,),
        in_specs=[
            pl.BlockSpec((RB, D), lambda i: (i, 0)),
            pl.BlockSpec((RB, D), lambda i: (i, 0)),
            pl.BlockSpec((2, RB, H), lambda i: (0, i, 0)),
            pl.BlockSpec((2, RB, H), lambda i: (0, i, 0)),
            pl.BlockSpec((NTILE, RB), lambda i: (0, i)),
            pl.BlockSpec((1, H), lambda i: (0, 0)),
            pl.BlockSpec((H, H), lambda i: (0, 0)),
            pl.BlockSpec((1, H), lambda i: (0, 0)),
            pl.BlockSpec((H, D), lambda i: (0, 0)),
            pl.BlockSpec((H, D), lambda i: (0, 0)),
            pl.BlockSpec((1, D), lambda i: (0, 0)),
        ],
        out_specs=pl.BlockSpec((RB, D), lambda i: (i, 0)),
        out_shape=jax.ShapeDtypeStruct((NPAD, D), jnp.float32),
    )(x, h, p2, aggp, denp, br2.reshape(1, H), Wo, bo.reshape(1, H),
      W2[:H], W2[H:], b2.reshape(1, D))


_GDN = lax.GatherDimensionNumbers(
    offset_dims=(), collapsed_slice_dims=(0,), start_index_map=(0,))


def _lane_shuffle(a, idx):
    return lax.gather(a, idx[:, None], _GDN, (1,),
                      mode=lax.GatherScatterMode.PROMISE_IN_BOUNDS)


def _lane_sum(a):
    for sh in (8, 4, 2, 1):
        idx = lax.iota(jnp.int32, 16) ^ sh
        a = a + _lane_shuffle(a, idx)
    return a


def _zero_rows(zb, nrows, ncolv):
    def row(r, _):
        for i in range(ncolv):
            zb[r, pl.ds(i * 16, 16)] = jnp.zeros((16,), jnp.float32)
        return 0
    lax.fori_loop(0, nrows, row, 0)


def _make_sc_sconv():
    mesh = plsc.VectorSubcoreMesh(core_axis_name="c", subcore_axis_name="s")

    @functools.partial(
        pl.kernel,
        mesh=mesh,
        out_type=jax.ShapeDtypeStruct((2, NPAD, H), jnp.float32),
        scratch_types=[
            pltpu.VMEM((56, CHS), jnp.int32),
            pltpu.VMEM((56, CHS), jnp.int32),
            pltpu.VMEM((CHS, H), jnp.float32),
            pltpu.VMEM((CHS, H), jnp.float32),
            pltpu.VMEM_SHARED((NPAD, H), jnp.float32),
            pltpu.SemaphoreType.DMA,
            pltpu.SemaphoreType.DMA,
            pltpu.SemaphoreType.DMA,
            pltpu.SemaphoreType.DMA,
            pltpu.SemaphoreType.DMA,
            pltpu.SemaphoreType.DMA,
        ],
    )
    def sconv_pass(y_hbm, gidx_hbm, didx_hbm, out_hbm,
                   gb, db, rows0, rows1, acc, sem0, sem1, ssem0, ssem1,
                   sem0b, sem1b):
        c = lax.axis_index("c")
        s = lax.axis_index("s")
        base = s * 80 + c * 48
        ncnt = jnp.where(c == 0, 48, 32)
        pltpu.sync_copy(gidx_hbm.at[pl.ds(base, 56)], gb)
        pltpu.sync_copy(didx_hbm.at[pl.ds(base, 56)], db)
        _zero_rows(rows0, CHS, H // 16)
        for t in range(640 // CHS):
            pltpu.sync_copy(rows0, acc.at[pl.ds(s * 640 + t * CHS, CHS)])
        plsc.subcore_barrier()

        bufs = (rows0, rows1)
        gsems = (sem0, sem1)
        gsemsb = (sem0b, sem1b)
        ssems = (ssem0, ssem1)
        HCH = CHS // 2

        def fire_gather(j, b):
            pltpu.async_copy(y_hbm.at[gb.at[j, pl.ds(0, HCH)]],
                             bufs[b].at[pl.ds(0, HCH)], gsems[b])
            pltpu.async_copy(y_hbm.at[gb.at[j, pl.ds(HCH, HCH)]],
                             bufs[b].at[pl.ds(HCH, HCH)], gsemsb[b])

        fire_gather(0, 0)

        def pair(g, _):
            for b in range(2):
                j = 2 * g + b

                @pl.when(j >= 1)
                def _():
                    pltpu.make_async_copy(
                        bufs[1 - b], acc.at[pl.ds(0, CHS)],
                        ssems[1 - b]).wait()

                @pl.when(j + 1 < ncnt)
                def _():
                    fire_gather(j + 1, 1 - b)
                pltpu.make_async_copy(
                    y_hbm.at[pl.ds(0, HCH)], bufs[b].at[pl.ds(0, HCH)],
                    gsems[b]).wait()
                pltpu.make_async_copy(
                    y_hbm.at[pl.ds(0, HCH)], bufs[b].at[pl.ds(HCH, HCH)],
                    gsemsb[b]).wait()
                pltpu.async_copy(bufs[b], acc.at[db.at[j]], ssems[b],
                                 add=True)
            return 0

        lax.fori_loop(0, ncnt // 2, pair, 0)
        pltpu.make_async_copy(bufs[1], acc.at[pl.ds(0, CHS)], ssems[1]).wait()
        plsc.subcore_barrier()
        pltpu.sync_copy(acc.at[pl.ds(s * 640, 640)],
                        out_hbm.at[c, pl.ds(s * 640, 640)])

    return sconv_pass


def _make_sc_attn():
    mesh = plsc.VectorSubcoreMesh(core_axis_name="c", subcore_axis_name="s")

    @functools.partial(
        pl.kernel,
        mesh=mesh,
        out_type=[jax.ShapeDtypeStruct((2, NPAD, H), jnp.float32),
                  jax.ShapeDtypeStruct((NTILE, NPAD), jnp.float32)],
        scratch_types=[
            pltpu.VMEM((NCHS, CHS), jnp.int32),
            pltpu.VMEM((NCHS, CHS), jnp.int32),
            pltpu.VMEM((1, 3 * CHA), jnp.int32),
            pltpu.VMEM((1, 3 * CHA), jnp.int32),
            pltpu.VMEM((1, CHA), jnp.int32),
            pltpu.VMEM((1, CHA), jnp.int32),
            pltpu.VMEM((3 * CHA, H), jnp.float32),
            pltpu.VMEM((3 * CHA, H), jnp.float32),
            pltpu.VMEM((NPAD,), jnp.float32),
            pltpu.VMEM_SHARED((NPAD, H), jnp.float32),
            pltpu.SemaphoreType.DMA,
            pltpu.SemaphoreType.DMA,
            pltpu.SemaphoreType.DMA,
            pltpu.SemaphoreType.DMA,
        ],
    )
    def attn_pass(t_hbm, sidx_hbm, didx_hbm, agg_hbm, den_hbm,
                  sb, db, ib0, ib1, ibd0, ibd1, qkvb0, qkvb1, dn, acc,
                  sem0, sem1, ssem0, ssem1):
        c = lax.axis_index("c")
        s = lax.axis_index("s")
        w = c * 16 + s
        pltpu.sync_copy(sidx_hbm.at[pl.ds(w * NCHS, NCHS)], sb)
        pltpu.sync_copy(didx_hbm.at[pl.ds(w * NCHS, NCHS)], db)
        _zero_rows(qkvb0, 80, H // 16)
        for t in range(640 // 80):
            pltpu.sync_copy(qkvb0.at[pl.ds(0, 80)],
                            acc.at[pl.ds(s * 640 + t * 80, 80)])

        def zdn(i, _):
            dn[pl.ds(i * 16, 16)] = jnp.zeros((16,), jnp.float32)
            return 0

        lax.fori_loop(0, NPAD // 16, zdn, 0)
        plsc.subcore_barrier()
        lane_iota = lax.iota(jnp.int32, 16)
        ibs = (ib0, ib1)
        ibds = (ibd0, ibd1)
        bufs = (qkvb0, qkvb1)
        sems = (sem0, sem1)
        ssems = (ssem0, ssem1)

        def fire(j, b):
            ib, ibd = ibs[b], ibds[b]
            jr = j >> 2
            col0 = (j & 3) * CHA
            for t in range(CHA // 16):
                dstv = db[jr, pl.ds(col0 + t * 16, 16)]
                srcv = sb[jr, pl.ds(col0 + t * 16, 16)]
                ibd[0, pl.ds(t * 16, 16)] = dstv
                ib[0, pl.ds(t * 16, 16)] = dstv
                ib[0, pl.ds(CHA + t * 16, 16)] = srcv + NPAD
                ib[0, pl.ds(2 * CHA + t * 16, 16)] = srcv + 2 * NPAD
            pltpu.async_copy(t_hbm.at[ib.at[0]], bufs[b], sems[b])

        fire(0, 0)

        def pair(g, _):
            for b in range(2):
                j = 2 * g + b

                @pl.when(j >= 1)
                def _():
                    pltpu.make_async_copy(
                        bufs[1 - b].at[pl.ds(2 * CHA, CHA)],
                        acc.at[pl.ds(0, CHA)], ssems[1 - b]).wait()

                @pl.when(j + 1 < NCHA)
                def _():
                    fire(j + 1, 1 - b)

                qkvb = bufs[b]
                ibd = ibds[b]
                pltpu.make_async_copy(
                    t_hbm.at[pl.ds(0, 3 * CHA)], qkvb, sems[b]).wait()

                def grp(gg, _):
                    dv16 = ibd[0, pl.ds(gg * 16, 16)]
                    for l in range(16):
                        ei = gg * 16 + l
                        a = (qkvb[ei, pl.ds(0, 16)]
                             * qkvb[CHA + ei, pl.ds(0, 16)])
                        for i in range(1, H // 16):
                            a = a + (qkvb[ei, pl.ds(i * 16, 16)]
                                     * qkvb[CHA + ei, pl.ds(i * 16, 16)])
                        ev = jnp.exp(_lane_sum(a))
                        dv = dv16[l]
                        base = (dv >> 4) << 4
                        lane = dv & 15
                        dn[pl.ds(base, 16)] = dn[pl.ds(base, 16)] + jnp.where(
                            lane_iota == lane, ev, 0.0)
                        for i in range(H // 16):
                            qkvb[2 * CHA + ei, pl.ds(i * 16, 16)] = (
                                qkvb[2 * CHA + ei, pl.ds(i * 16, 16)] * ev)
                    return 0

                lax.fori_loop(0, CHA // 16, grp, 0)
                pltpu.async_copy(qkvb.at[pl.ds(2 * CHA, CHA)],
                                 acc.at[ibd.at[0]], ssems[b], add=True)
            return 0

        lax.fori_loop(0, NCHA // 2, pair, 0)
        pltpu.make_async_copy(bufs[1].at[pl.ds(2 * CHA, CHA)],
                              acc.at[pl.ds(0, CHA)], ssems[1]).wait()
        plsc.subcore_barrier()
        pltpu.sync_copy(acc.at[pl.ds(s * 640, 640)],
                        agg_hbm.at[c, pl.ds(s * 640, 640)])
        pltpu.sync_copy(dn, den_hbm.at[w])

    return attn_pass


_make_sc_sconv = functools.cache(_make_sc_sconv)
_make_sc_attn = functools.cache(_make_sc_attn)


def _sc_sconv(y, gidx, didx):
    return _make_sc_sconv()(y, gidx, didx)


def _sc_attn(tqkv, sidx, didx):
    return _make_sc_attn()(tqkv, sidx, didx)


def kernel(x, edge_index, kernel_offsets, W1, b1, W2, b2, Wr1, br1, Wr2, br2,
           Wq, bq, Wk, bk, Wv, bv, Wo, bo):
    src = edge_index[0]
    dst = edge_index[1]
    npad_e = EPAD - E
    src_p = jnp.concatenate([src, jnp.zeros((npad_e,), jnp.int32)])
    koff_p = jnp.concatenate([kernel_offsets,
                              jnp.zeros((npad_e,), jnp.int32)])
    dst_p = jnp.concatenate([dst, jnp.full((npad_e,), N, jnp.int32)])
    gidx = (koff_p * NPAD + src_p).reshape(EPAD // CHS, CHS)
    didx_s = dst_p.reshape(EPAD // CHS, CHS)
    sidx = src_p.reshape(EPAD // CHS, CHS)

    Wcat1 = jnp.transpose(Wr1, (1, 0, 2)).reshape(H, K * H)
    Wcat2 = jnp.transpose(Wr2, (1, 0, 2)).reshape(H, K * H)

    inv_s = 1.0 / jnp.sqrt(jnp.float32(H))
    Wqkv = jnp.stack([Wq * inv_s, Wk, Wv])
    bqkv = jnp.stack([bq * inv_s, bk, bv]).reshape(3, 1, H)

    x_p = jnp.pad(x, ((0, NPAD - N), (0, 0)))
    h = _conv1(x_p, W1, b1)

    y1 = _taps_from_h(h, Wcat1).reshape(K * NPAD, H)
    p1 = _sc_sconv(y1, gidx, didx_s)
    y2 = _mid(p1, br1, Wcat2).reshape(K * NPAD, H)
    p2 = _sc_sconv(y2, gidx, didx_s)

    qkv = _qkv(h, Wqkv, bqkv)
    aggp, denp = _sc_attn(qkv.reshape(3 * NPAD, H), sidx, didx_s)

    return _final(x_p, h, p2, aggp, denp, br2, Wo, bo, W2, b2)[:N]


# speedup vs baseline: 1.0127x; 1.0127x over previous
import functools

import jax
import jax.numpy as jnp
from jax import lax
from jax.experimental import pallas as pl
from jax.experimental.pallas import tpu as pltpu
from jax.experimental.pallas import tpu_sc as plsc

N = 10000
E = 160000
D = 256
K = 27
H = 128

NPAD = 10240
EPAD = 163840
NTILE = 32
EPT = EPAD // NTILE
CHS = 128
NCHS = EPT // CHS
CHA = 32
NCHA = EPT // CHA
RB = 2048
NRB = NPAD // RB


def _mm_body(x_ref, w_ref, b_ref, o_ref):
    o_ref[...] = (
        jnp.dot(x_ref[...], w_ref[...], preferred_element_type=jnp.float32)
        + b_ref[...]
    )


def _conv1(x, W1, b1):
    return pl.pallas_call(
        _mm_body,
        grid=(NRB,),
        in_specs=[
            pl.BlockSpec((RB, D), lambda i: (i, 0)),
            pl.BlockSpec((D, D), lambda i: (0, 0)),
            pl.BlockSpec((1, D), lambda i: (0, 0)),
        ],
        out_specs=pl.BlockSpec((RB, D), lambda i: (i, 0)),
        out_shape=jax.ShapeDtypeStruct((NPAD, D), jnp.float32),
    )(x, W1, b1.reshape(1, D))


def _qkv_body(h_ref, w_ref, b_ref, o_ref):
    tx = h_ref[:, H:]
    o_ref[0] = (
        jnp.dot(tx, w_ref[0], preferred_element_type=jnp.float32) + b_ref[0]
    )


def _qkv(h, Wqkv, bqkv):
    return pl.pallas_call(
        _qkv_body,
        grid=(3, NRB),
        in_specs=[
            pl.BlockSpec((RB, D), lambda p, i: (i, 0)),
            pl.BlockSpec((1, H, H), lambda p, i: (p, 0, 0)),
            pl.BlockSpec((1, 1, H), lambda p, i: (p, 0, 0)),
        ],
        out_specs=pl.BlockSpec((1, RB, H), lambda p, i: (p, i, 0)),
        out_shape=jax.ShapeDtypeStruct((3, NPAD, H), jnp.float32),
    )(h, Wqkv, bqkv)


def _taps_body(f_ref, w_ref, o_ref):
    o_ref[...] = jnp.dot(
        f_ref[...], w_ref[...], preferred_element_type=jnp.float32
    )


def _taps_from_h(h, Wcat):
    def body(h_ref, w_ref, o_ref):
        o_ref[0] = jnp.dot(
            h_ref[:, :H], w_ref[...], preferred_element_type=jnp.float32
        )

    return pl.pallas_call(
        body,
        grid=(K, NRB),
        in_specs=[
            pl.BlockSpec((RB, D), lambda k, i: (i, 0)),
            pl.BlockSpec((H, H), lambda k, i: (0, k)),
        ],
        out_specs=pl.BlockSpec((1, RB, H), lambda k, i: (k, i, 0)),
        out_shape=jax.ShapeDtypeStruct((K, NPAD, H), jnp.float32),
    )(h, Wcat)


def _mid_body(p_ref, b_ref, w_ref, o_ref):
    f2 = jnp.maximum(p_ref[0] + p_ref[1] + b_ref[...], 0.0)
    o_ref[0] = jnp.dot(f2, w_ref[...], preferred_element_type=jnp.float32)


def _mid(p1, br1, Wcat2):
    return pl.pallas_call(
        _mid_body,
        grid=(K, NRB),
        in_specs=[
            pl.BlockSpec((2, RB, H), lambda k, i: (0, i, 0)),
            pl.BlockSpec((1, H), lambda k, i: (0, 0)),
            pl.BlockSpec((H, H), lambda k, i: (0, k)),
        ],
        out_specs=pl.BlockSpec((1, RB, H), lambda k, i: (k, i, 0)),
        out_shape=jax.ShapeDtypeStruct((K, NPAD, H), jnp.float32),
    )(p1, br1.reshape(1, H), Wcat2)


def _final_body(x_ref, h_ref, p2_ref, ag_ref, dn_ref, br2_ref, wo_ref, bo_ref,
                w2a_ref, w2b_ref, b2_ref, o_ref):
    conv_x = h_ref[:, :H]
    trans_x = h_ref[:, H:]
    out2 = jnp.maximum(p2_ref[0] + p2_ref[1] + br2_ref[...], 0.0)
    cx2 = out2 + 2.0 * conv_x
    ag = ag_ref[0] + ag_ref[1]
    denom = jnp.sum(dn_ref[...], axis=0)[:, None] + 1e-9
    agg = ag / denom
    tr = (
        jnp.dot(agg, wo_ref[...], preferred_element_type=jnp.float32)
        + bo_ref[...] + trans_x
    )
    res = (
        jnp.dot(cx2, w2a_ref[...], preferred_element_type=jnp.float32)
        + jnp.dot(tr, w2b_ref[...], preferred_element_type=jnp.float32)
        + b2_ref[...]
    )
    o_ref[...] = x_ref[...] + res


def _final(x, h, p2, aggp, denp, br2, Wo, bo, W2, b2):
    return pl.pallas_call(
        _final_body,
        grid=(NRB,),
        in_specs=[
            pl.BlockSpec((RB, D), lambda i: (i, 0)),
            pl.BlockSpec((RB, D), lambda i: (i, 0)),
            pl.BlockSpec((2, RB, H), lambda i: (0, i, 0)),
            pl.BlockSpec((2, RB, H), lambda i: (0, i, 0)),
            pl.BlockSpec((NTILE, RB), lambda i: (0, i)),
            pl.BlockSpec((1, H), lambda i: (0, 0)),
            pl.BlockSpec((H, H), lambda i: (0, 0)),
            pl.BlockSpec((1, H), lambda i: (0, 0)),
            pl.BlockSpec((H, D), lambda i: (0, 0)),
            pl.BlockSpec((H, D), lambda i: (0, 0)),
            pl.BlockSpec((1, D), lambda i: (0, 0)),
        ],
        out_specs=pl.BlockSpec((RB, D), lambda i: (i, 0)),
        out_shape=jax.ShapeDtypeStruct((NPAD, D), jnp.float32),
    )(x, h, p2, aggp, denp, br2.reshape(1, H), Wo, bo.reshape(1, H),
      W2[:H], W2[H:], b2.reshape(1, D))


_GDN = lax.GatherDimensionNumbers(
    offset_dims=(), collapsed_slice_dims=(0,), start_index_map=(0,))


def _lane_shuffle(a, idx):
    return lax.gather(a, idx[:, None], _GDN, (1,),
                      mode=lax.GatherScatterMode.PROMISE_IN_BOUNDS)


def _lane_sum(a):
    for sh in (8, 4, 2, 1):
        idx = lax.iota(jnp.int32, 16) ^ sh
        a = a + _lane_shuffle(a, idx)
    return a


def _zero_rows(zb, nrows, ncolv):
    def row(r, _):
        for i in range(ncolv):
            zb[r, pl.ds(i * 16, 16)] = jnp.zeros((16,), jnp.float32)
        return 0
    lax.fori_loop(0, nrows, row, 0)


def _make_sc_sconv():
    mesh = plsc.VectorSubcoreMesh(core_axis_name="c", subcore_axis_name="s")

    @functools.partial(
        pl.kernel,
        mesh=mesh,
        out_type=jax.ShapeDtypeStruct((2, NPAD, H), jnp.float32),
        scratch_types=[
            pltpu.VMEM((56, CHS), jnp.int32),
            pltpu.VMEM((56, CHS), jnp.int32),
            pltpu.VMEM((CHS, H), jnp.float32),
            pltpu.VMEM((CHS, H), jnp.float32),
            pltpu.VMEM_SHARED((NPAD, H), jnp.float32),
            pltpu.SemaphoreType.DMA,
            pltpu.SemaphoreType.DMA,
            pltpu.SemaphoreType.DMA,
            pltpu.SemaphoreType.DMA,
            pltpu.SemaphoreType.DMA,
            pltpu.SemaphoreType.DMA,
        ],
    )
    def sconv_pass(y_hbm, gidx_hbm, didx_hbm, out_hbm,
                   gb, db, rows0, rows1, acc, sem0, sem1, ssem0, ssem1,
                   sem0b, sem1b):
        c = lax.axis_index("c")
        s = lax.axis_index("s")
        base = s * 80 + c * 56
        ncnt = jnp.where(c == 0, 56, 24)
        pltpu.sync_copy(gidx_hbm.at[pl.ds(base, 56)], gb)
        pltpu.sync_copy(didx_hbm.at[pl.ds(base, 56)], db)
        _zero_rows(rows0, CHS, H // 16)
        for t in range(640 // CHS):
            pltpu.sync_copy(rows0, acc.at[pl.ds(s * 640 + t * CHS, CHS)])
        plsc.subcore_barrier()

        bufs = (rows0, rows1)
        gsems = (sem0, sem1)
        gsemsb = (sem0b, sem1b)
        ssems = (ssem0, ssem1)
        HCH = CHS // 2

        def fire_gather(j, b):
            pltpu.async_copy(y_hbm.at[gb.at[j, pl.ds(0, HCH)]],
                             bufs[b].at[pl.ds(0, HCH)], gsems[b])
            pltpu.async_copy(y_hbm.at[gb.at[j, pl.ds(HCH, HCH)]],
                             bufs[b].at[pl.ds(HCH, HCH)], gsemsb[b])

        fire_gather(0, 0)

        def pair(g, _):
            for b in range(2):
                j = 2 * g + b

                @pl.when(j >= 1)
                def _():
                    pltpu.make_async_copy(
                        bufs[1 - b], acc.at[pl.ds(0, CHS)],
                        ssems[1 - b]).wait()

                @pl.when(j + 1 < ncnt)
                def _():
                    fire_gather(j + 1, 1 - b)
                pltpu.make_async_copy(
                    y_hbm.at[pl.ds(0, HCH)], bufs[b].at[pl.ds(0, HCH)],
                    gsems[b]).wait()
                pltpu.make_async_copy(
                    y_hbm.at[pl.ds(0, HCH)], bufs[b].at[pl.ds(HCH, HCH)],
                    gsemsb[b]).wait()
                pltpu.async_copy(bufs[b], acc.at[db.at[j]], ssems[b],
                                 add=True)
            return 0

        lax.fori_loop(0, ncnt // 2, pair, 0)
        pltpu.make_async_copy(bufs[1], acc.at[pl.ds(0, CHS)], ssems[1]).wait()
        plsc.subcore_barrier()
        pltpu.sync_copy(acc.at[pl.ds(s * 640, 640)],
                        out_hbm.at[c, pl.ds(s * 640, 640)])

    return sconv_pass


def _make_sc_attn():
    mesh = plsc.VectorSubcoreMesh(core_axis_name="c", subcore_axis_name="s")

    @functools.partial(
        pl.kernel,
        mesh=mesh,
        out_type=[jax.ShapeDtypeStruct((2, NPAD, H), jnp.float32),
                  jax.ShapeDtypeStruct((NTILE, NPAD), jnp.float32)],
        scratch_types=[
            pltpu.VMEM((48, CHS), jnp.int32),
            pltpu.VMEM((48, CHS), jnp.int32),
            pltpu.VMEM((1, 3 * CHA), jnp.int32),
            pltpu.VMEM((1, 3 * CHA), jnp.int32),
            pltpu.VMEM((1, CHA), jnp.int32),
            pltpu.VMEM((1, CHA), jnp.int32),
            pltpu.VMEM((3 * CHA, H), jnp.float32),
            pltpu.VMEM((3 * CHA, H), jnp.float32),
            pltpu.VMEM((NPAD,), jnp.float32),
            pltpu.VMEM_SHARED((NPAD, H), jnp.float32),
            pltpu.SemaphoreType.DMA,
            pltpu.SemaphoreType.DMA,
            pltpu.SemaphoreType.DMA,
            pltpu.SemaphoreType.DMA,
        ],
    )
    def attn_pass(t_hbm, sidx_hbm, didx_hbm, agg_hbm, den_hbm,
                  sb, db, ib0, ib1, ibd0, ibd1, qkvb0, qkvb1, dn, acc,
                  sem0, sem1, ssem0, ssem1):
        c = lax.axis_index("c")
        s = lax.axis_index("s")
        w = c * 16 + s
        abase = s * 80 + c * 48
        acnt = jnp.where(c == 0, 48 * 4, 32 * 4)
        pltpu.sync_copy(sidx_hbm.at[pl.ds(abase, 48)], sb)
        pltpu.sync_copy(didx_hbm.at[pl.ds(abase, 48)], db)
        _zero_rows(qkvb0, 80, H // 16)
        for t in range(640 // 80):
            pltpu.sync_copy(qkvb0.at[pl.ds(0, 80)],
                            acc.at[pl.ds(s * 640 + t * 80, 80)])

        def zdn(i, _):
            dn[pl.ds(i * 16, 16)] = jnp.zeros((16,), jnp.float32)
            return 0

        lax.fori_loop(0, NPAD // 16, zdn, 0)
        plsc.subcore_barrier()
        lane_iota = lax.iota(jnp.int32, 16)
        ibs = (ib0, ib1)
        ibds = (ibd0, ibd1)
        bufs = (qkvb0, qkvb1)
        sems = (sem0, sem1)
        ssems = (ssem0, ssem1)

        def fire(j, b):
            ib, ibd = ibs[b], ibds[b]
            jr = j >> 2
            col0 = (j & 3) * CHA
            for t in range(CHA // 16):
                dstv = db[jr, pl.ds(col0 + t * 16, 16)]
                srcv = sb[jr, pl.ds(col0 + t * 16, 16)]
                ibd[0, pl.ds(t * 16, 16)] = dstv
                ib[0, pl.ds(t * 16, 16)] = dstv
                ib[0, pl.ds(CHA + t * 16, 16)] = srcv + NPAD
                ib[0, pl.ds(2 * CHA + t * 16, 16)] = srcv + 2 * NPAD
            pltpu.async_copy(t_hbm.at[ib.at[0]], bufs[b], sems[b])

        fire(0, 0)

        def pair(g, _):
            for b in range(2):
                j = 2 * g + b

                @pl.when(j >= 1)
                def _():
                    pltpu.make_async_copy(
                        bufs[1 - b].at[pl.ds(2 * CHA, CHA)],
                        acc.at[pl.ds(0, CHA)], ssems[1 - b]).wait()

                @pl.when(j + 1 < acnt)
                def _():
                    fire(j + 1, 1 - b)

                qkvb = bufs[b]
                ibd = ibds[b]
                pltpu.make_async_copy(
                    t_hbm.at[pl.ds(0, 3 * CHA)], qkvb, sems[b]).wait()

                def grp(gg, _):
                    dv16 = ibd[0, pl.ds(gg * 16, 16)]
                    for l in range(16):
                        ei = gg * 16 + l
                        a = (qkvb[ei, pl.ds(0, 16)]
                             * qkvb[CHA + ei, pl.ds(0, 16)])
                        for i in range(1, H // 16):
                            a = a + (qkvb[ei, pl.ds(i * 16, 16)]
                                     * qkvb[CHA + ei, pl.ds(i * 16, 16)])
                        ev = jnp.exp(_lane_sum(a))
                        dv = dv16[l]
                        base = (dv >> 4) << 4
                        lane = dv & 15
                        dn[pl.ds(base, 16)] = dn[pl.ds(base, 16)] + jnp.where(
                            lane_iota == lane, ev, 0.0)
                        for i in range(H // 16):
                            qkvb[2 * CHA + ei, pl.ds(i * 16, 16)] = (
                                qkvb[2 * CHA + ei, pl.ds(i * 16, 16)] * ev)
                    return 0

                lax.fori_loop(0, CHA // 16, grp, 0)
                pltpu.async_copy(qkvb.at[pl.ds(2 * CHA, CHA)],
                                 acc.at[ibd.at[0]], ssems[b], add=True)
            return 0

        lax.fori_loop(0, acnt // 2, pair, 0)
        pltpu.make_async_copy(bufs[1].at[pl.ds(2 * CHA, CHA)],
                              acc.at[pl.ds(0, CHA)], ssems[1]).wait()
        plsc.subcore_barrier()
        pltpu.sync_copy(acc.at[pl.ds(s * 640, 640)],
                        agg_hbm.at[c, pl.ds(s * 640, 640)])
        pltpu.sync_copy(dn, den_hbm.at[w])

    return attn_pass


_make_sc_sconv = functools.cache(_make_sc_sconv)
_make_sc_attn = functools.cache(_make_sc_attn)


def _sc_sconv(y, gidx, didx):
    return _make_sc_sconv()(y, gidx, didx)


def _sc_attn(tqkv, sidx, didx):
    return _make_sc_attn()(tqkv, sidx, didx)


def kernel(x, edge_index, kernel_offsets, W1, b1, W2, b2, Wr1, br1, Wr2, br2,
           Wq, bq, Wk, bk, Wv, bv, Wo, bo):
    src = edge_index[0]
    dst = edge_index[1]
    npad_e = EPAD - E
    src_p = jnp.concatenate([src, jnp.zeros((npad_e,), jnp.int32)])
    koff_p = jnp.concatenate([kernel_offsets,
                              jnp.zeros((npad_e,), jnp.int32)])
    dst_p = jnp.concatenate([dst, jnp.full((npad_e,), N, jnp.int32)])
    gidx = (koff_p * NPAD + src_p).reshape(EPAD // CHS, CHS)
    didx_s = dst_p.reshape(EPAD // CHS, CHS)
    sidx = src_p.reshape(EPAD // CHS, CHS)

    Wcat1 = jnp.transpose(Wr1, (1, 0, 2)).reshape(H, K * H)
    Wcat2 = jnp.transpose(Wr2, (1, 0, 2)).reshape(H, K * H)

    inv_s = 1.0 / jnp.sqrt(jnp.float32(H))
    Wqkv = jnp.stack([Wq * inv_s, Wk, Wv])
    bqkv = jnp.stack([bq * inv_s, bk, bv]).reshape(3, 1, H)

    x_p = jnp.pad(x, ((0, NPAD - N), (0, 0)))
    h = _conv1(x_p, W1, b1)

    y1 = _taps_from_h(h, Wcat1).reshape(K * NPAD, H)
    p1 = _sc_sconv(y1, gidx, didx_s)
    y2 = _mid(p1, br1, Wcat2).reshape(K * NPAD, H)
    p2 = _sc_sconv(y2, gidx, didx_s)

    qkv = _qkv(h, Wqkv, bqkv)
    aggp, denp = _sc_attn(qkv.reshape(3 * NPAD, H), sidx, didx_s)

    return _final(x_p, h, p2, aggp, denp, br2, Wo, bo, W2, b2)[:N]


# speedup vs baseline: 1.0171x; 1.0044x over previous
import functools

import jax
import jax.numpy as jnp
from jax import lax
from jax.experimental import pallas as pl
from jax.experimental.pallas import tpu as pltpu
from jax.experimental.pallas import tpu_sc as plsc

N = 10000
E = 160000
D = 256
K = 27
H = 128

NPAD = 10240
EPAD = 163840
NTILE = 32
EPT = EPAD // NTILE
CHS = 128
NCHS = EPT // CHS
CHA = 32
NCHA = EPT // CHA
RB = 2048
NRB = NPAD // RB


def _mm_body(x_ref, w_ref, b_ref, o_ref):
    o_ref[...] = (
        jnp.dot(x_ref[...], w_ref[...], preferred_element_type=jnp.float32)
        + b_ref[...]
    )


def _conv1(x, W1, b1):
    return pl.pallas_call(
        _mm_body,
        grid=(NRB,),
        in_specs=[
            pl.BlockSpec((RB, D), lambda i: (i, 0)),
            pl.BlockSpec((D, D), lambda i: (0, 0)),
            pl.BlockSpec((1, D), lambda i: (0, 0)),
        ],
        out_specs=pl.BlockSpec((RB, D), lambda i: (i, 0)),
        out_shape=jax.ShapeDtypeStruct((NPAD, D), jnp.float32),
    )(x, W1, b1.reshape(1, D))


def _qkv_body(h_ref, w_ref, b_ref, o_ref):
    tx = h_ref[:, H:]
    o_ref[0] = (
        jnp.dot(tx, w_ref[0], preferred_element_type=jnp.float32) + b_ref[0]
    )


def _qkv(h, Wqkv, bqkv):
    return pl.pallas_call(
        _qkv_body,
        grid=(3, NRB),
        in_specs=[
            pl.BlockSpec((RB, D), lambda p, i: (i, 0)),
            pl.BlockSpec((1, H, H), lambda p, i: (p, 0, 0)),
            pl.BlockSpec((1, 1, H), lambda p, i: (p, 0, 0)),
        ],
        out_specs=pl.BlockSpec((1, RB, H), lambda p, i: (p, i, 0)),
        out_shape=jax.ShapeDtypeStruct((3, NPAD, H), jnp.float32),
    )(h, Wqkv, bqkv)


def _taps_body(f_ref, w_ref, o_ref):
    o_ref[...] = jnp.dot(
        f_ref[...], w_ref[...], preferred_element_type=jnp.float32
    )


def _taps_from_h(h, Wcat):
    def body(h_ref, w_ref, o_ref):
        o_ref[0] = jnp.dot(
            h_ref[:, :H], w_ref[...], preferred_element_type=jnp.float32
        )

    return pl.pallas_call(
        body,
        grid=(K, NRB),
        in_specs=[
            pl.BlockSpec((RB, D), lambda k, i: (i, 0)),
            pl.BlockSpec((H, H), lambda k, i: (0, k)),
        ],
        out_specs=pl.BlockSpec((1, RB, H), lambda k, i: (k, i, 0)),
        out_shape=jax.ShapeDtypeStruct((K, NPAD, H), jnp.float32),
    )(h, Wcat)


def _mid_body(p_ref, b_ref, w_ref, o_ref):
    f2 = jnp.maximum(p_ref[0] + p_ref[1] + b_ref[...], 0.0)
    o_ref[0] = jnp.dot(f2, w_ref[...], preferred_element_type=jnp.float32)


def _mid(p1, br1, Wcat2):
    return pl.pallas_call(
        _mid_body,
        grid=(K, NRB),
        in_specs=[
            pl.BlockSpec((2, RB, H), lambda k, i: (0, i, 0)),
            pl.BlockSpec((1, H), lambda k, i: (0, 0)),
            pl.BlockSpec((H, H), lambda k, i: (0, k)),
        ],
        out_specs=pl.BlockSpec((1, RB, H), lambda k, i: (k, i, 0)),
        out_shape=jax.ShapeDtypeStruct((K, NPAD, H), jnp.float32),
    )(p1, br1.reshape(1, H), Wcat2)


def _final_body(x_ref, h_ref, p2_ref, ag_ref, dn_ref, br2_ref, wo_ref, bo_ref,
                w2a_ref, w2b_ref, b2_ref, o_ref):
    conv_x = h_ref[:, :H]
    trans_x = h_ref[:, H:]
    out2 = jnp.maximum(p2_ref[0] + p2_ref[1] + br2_ref[...], 0.0)
    cx2 = out2 + 2.0 * conv_x
    ag = ag_ref[0] + ag_ref[1]
    denom = jnp.sum(dn_ref[...], axis=0)[:, None] + 1e-9
    agg = ag / denom
    tr = (
        jnp.dot(agg, wo_ref[...], preferred_element_type=jnp.float32)
        + bo_ref[...] + trans_x
    )
    res = (
        jnp.dot(cx2, w2a_ref[...], preferred_element_type=jnp.float32)
        + jnp.dot(tr, w2b_ref[...], preferred_element_type=jnp.float32)
        + b2_ref[...]
    )
    o_ref[...] = x_ref[...] + res


def _final(x, h, p2, aggp, denp, br2, Wo, bo, W2, b2):
    return pl.pallas_call(
        _final_body,
        grid=(NRB,),
        in_specs=[
            pl.BlockSpec((RB, D), lambda i: (i, 0)),
            pl.BlockSpec((RB, D), lambda i: (i, 0)),
            pl.BlockSpec((2, RB, H), lambda i: (0, i, 0)),
            pl.BlockSpec((2, RB, H), lambda i: (0, i, 0)),
            pl.BlockSpec((NTILE, RB), lambda i: (0, i)),
            pl.BlockSpec((1, H), lambda i: (0, 0)),
            pl.BlockSpec((H, H), lambda i: (0, 0)),
            pl.BlockSpec((1, H), lambda i: (0, 0)),
            pl.BlockSpec((H, D), lambda i: (0, 0)),
            pl.BlockSpec((H, D), lambda i: (0, 0)),
            pl.BlockSpec((1, D), lambda i: (0, 0)),
        ],
        out_specs=pl.BlockSpec((RB, D), lambda i: (i, 0)),
        out_shape=jax.ShapeDtypeStruct((NPAD, D), jnp.float32),
    )(x, h, p2, aggp, denp, br2.reshape(1, H), Wo, bo.reshape(1, H),
      W2[:H], W2[H:], b2.reshape(1, D))


_GDN = lax.GatherDimensionNumbers(
    offset_dims=(), collapsed_slice_dims=(0,), start_index_map=(0,))


def _lane_shuffle(a, idx):
    return lax.gather(a, idx[:, None], _GDN, (1,),
                      mode=lax.GatherScatterMode.PROMISE_IN_BOUNDS)


def _lane_sum(a):
    for sh in (8, 4, 2, 1):
        idx = lax.iota(jnp.int32, 16) ^ sh
        a = a + _lane_shuffle(a, idx)
    return a


def _zero_rows(zb, nrows, ncolv):
    def row(r, _):
        for i in range(ncolv):
            zb[r, pl.ds(i * 16, 16)] = jnp.zeros((16,), jnp.float32)
        return 0
    lax.fori_loop(0, nrows, row, 0)


def _make_sc_sconv():
    mesh = plsc.VectorSubcoreMesh(core_axis_name="c", subcore_axis_name="s")

    @functools.partial(
        pl.kernel,
        mesh=mesh,
        out_type=jax.ShapeDtypeStruct((2, NPAD, H), jnp.float32),
        scratch_types=[
            pltpu.VMEM((56, CHS), jnp.int32),
            pltpu.VMEM((56, CHS), jnp.int32),
            pltpu.VMEM((CHS, H), jnp.float32),
            pltpu.VMEM((CHS, H), jnp.float32),
            pltpu.VMEM_SHARED((NPAD, H), jnp.float32),
            pltpu.SemaphoreType.DMA,
            pltpu.SemaphoreType.DMA,
            pltpu.SemaphoreType.DMA,
            pltpu.SemaphoreType.DMA,
            pltpu.SemaphoreType.DMA,
            pltpu.SemaphoreType.DMA,
        ],
    )
    def sconv_pass(y_hbm, gidx_hbm, didx_hbm, out_hbm,
                   gb, db, rows0, rows1, acc, sem0, sem1, ssem0, ssem1,
                   sem0b, sem1b):
        c = lax.axis_index("c")
        s = lax.axis_index("s")
        base = s * 80 + c * 56
        ncnt = jnp.where(c == 0, 56, 24)
        pltpu.sync_copy(gidx_hbm.at[pl.ds(base, 56)], gb)
        pltpu.sync_copy(didx_hbm.at[pl.ds(base, 56)], db)
        _zero_rows(rows0, CHS, H // 16)
        for t in range(640 // CHS):
            pltpu.sync_copy(rows0, acc.at[pl.ds(s * 640 + t * CHS, CHS)])
        plsc.subcore_barrier()

        bufs = (rows0, rows1)
        gsems = (sem0, sem1)
        gsemsb = (sem0b, sem1b)
        ssems = (ssem0, ssem1)
        HCH = CHS // 2

        def fire_gather(j, b):
            pltpu.async_copy(y_hbm.at[gb.at[j, pl.ds(0, HCH)]],
                             bufs[b].at[pl.ds(0, HCH)], gsems[b])
            pltpu.async_copy(y_hbm.at[gb.at[j, pl.ds(HCH, HCH)]],
                             bufs[b].at[pl.ds(HCH, HCH)], gsemsb[b])

        fire_gather(0, 0)

        def pair(g, _):
            for b in range(2):
                j = 2 * g + b

                @pl.when(j >= 1)
                def _():
                    pltpu.make_async_copy(
                        bufs[1 - b], acc.at[pl.ds(0, CHS)],
                        ssems[1 - b]).wait()

                @pl.when(j + 1 < ncnt)
                def _():
                    fire_gather(j + 1, 1 - b)
                pltpu.make_async_copy(
                    y_hbm.at[pl.ds(0, HCH)], bufs[b].at[pl.ds(0, HCH)],
                    gsems[b]).wait()
                pltpu.make_async_copy(
                    y_hbm.at[pl.ds(0, HCH)], bufs[b].at[pl.ds(HCH, HCH)],
                    gsemsb[b]).wait()
                pltpu.async_copy(bufs[b], acc.at[db.at[j]], ssems[b],
                                 add=True)
            return 0

        lax.fori_loop(0, ncnt // 2, pair, 0)
        pltpu.make_async_copy(bufs[1], acc.at[pl.ds(0, CHS)], ssems[1]).wait()
        plsc.subcore_barrier()
        pltpu.sync_copy(acc.at[pl.ds(s * 640, 640)],
                        out_hbm.at[c, pl.ds(s * 640, 640)])

    return sconv_pass


def _make_sc_attn():
    mesh = plsc.VectorSubcoreMesh(core_axis_name="c", subcore_axis_name="s")

    @functools.partial(
        pl.kernel,
        mesh=mesh,
        out_type=[jax.ShapeDtypeStruct((2, NPAD, H), jnp.float32),
                  jax.ShapeDtypeStruct((NTILE, NPAD), jnp.float32)],
        scratch_types=[
            pltpu.VMEM((NCHS, CHS), jnp.int32),
            pltpu.VMEM((NCHS, CHS), jnp.int32),
            pltpu.VMEM((1, 3 * CHA), jnp.int32),
            pltpu.VMEM((1, 3 * CHA), jnp.int32),
            pltpu.VMEM((1, CHA), jnp.int32),
            pltpu.VMEM((1, CHA), jnp.int32),
            pltpu.VMEM((3 * CHA, H), jnp.float32),
            pltpu.VMEM((3 * CHA, H), jnp.float32),
            pltpu.VMEM((NPAD,), jnp.float32),
            pltpu.VMEM_SHARED((NPAD, H), jnp.float32),
            pltpu.SemaphoreType.DMA,
            pltpu.SemaphoreType.DMA,
            pltpu.SemaphoreType.DMA,
            pltpu.SemaphoreType.DMA,
        ],
    )
    def attn_pass(t_hbm, sidx_hbm, didx_hbm, agg_hbm, den_hbm,
                  sb, db, ib0, ib1, ibd0, ibd1, qkvb0, qkvb1, dn, acc,
                  sem0, sem1, ssem0, ssem1):
        c = lax.axis_index("c")
        s = lax.axis_index("s")
        w = c * 16 + s
        pltpu.sync_copy(sidx_hbm.at[pl.ds(w * NCHS, NCHS)], sb)
        pltpu.sync_copy(didx_hbm.at[pl.ds(w * NCHS, NCHS)], db)
        _zero_rows(qkvb0, 80, H // 16)
        for t in range(640 // 80):
            pltpu.sync_copy(qkvb0.at[pl.ds(0, 80)],
                            acc.at[pl.ds(s * 640 + t * 80, 80)])

        def zdn(i, _):
            dn[pl.ds(i * 16, 16)] = jnp.zeros((16,), jnp.float32)
            return 0

        lax.fori_loop(0, NPAD // 16, zdn, 0)
        plsc.subcore_barrier()
        lane_iota = lax.iota(jnp.int32, 16)
        ibs = (ib0, ib1)
        ibds = (ibd0, ibd1)
        bufs = (qkvb0, qkvb1)
        sems = (sem0, sem1)
        ssems = (ssem0, ssem1)

        def fire(j, b):
            ib, ibd = ibs[b], ibds[b]
            jr = j >> 2
            col0 = (j & 3) * CHA
            for t in range(CHA // 16):
                dstv = db[jr, pl.ds(col0 + t * 16, 16)]
                srcv = sb[jr, pl.ds(col0 + t * 16, 16)]
                ibd[0, pl.ds(t * 16, 16)] = dstv
                ib[0, pl.ds(t * 16, 16)] = dstv
                ib[0, pl.ds(CHA + t * 16, 16)] = srcv + NPAD
                ib[0, pl.ds(2 * CHA + t * 16, 16)] = srcv + 2 * NPAD
            pltpu.async_copy(t_hbm.at[ib.at[0]], bufs[b], sems[b])

        fire(0, 0)

        def pair(g, _):
            for b in range(2):
                j = 2 * g + b

                @pl.when(j >= 1)
                def _():
                    pltpu.make_async_copy(
                        bufs[1 - b].at[pl.ds(2 * CHA, CHA)],
                        acc.at[pl.ds(0, CHA)], ssems[1 - b]).wait()

                @pl.when(j + 1 < NCHA)
                def _():
                    fire(j + 1, 1 - b)

                qkvb = bufs[b]
                ibd = ibds[b]
                pltpu.make_async_copy(
                    t_hbm.at[pl.ds(0, 3 * CHA)], qkvb, sems[b]).wait()

                def grp(gg, _):
                    dv16 = ibd[0, pl.ds(gg * 16, 16)]
                    for l in range(16):
                        ei = gg * 16 + l
                        a = (qkvb[ei, pl.ds(0, 16)]
                             * qkvb[CHA + ei, pl.ds(0, 16)])
                        for i in range(1, H // 16):
                            a = a + (qkvb[ei, pl.ds(i * 16, 16)]
                                     * qkvb[CHA + ei, pl.ds(i * 16, 16)])
                        ev = jnp.exp(_lane_sum(a))
                        dv = dv16[l]
                        base = (dv >> 4) << 4
                        lane = dv & 15
                        dn[pl.ds(base, 16)] = dn[pl.ds(base, 16)] + jnp.where(
                            lane_iota == lane, ev, 0.0)
                        for i in range(H // 16):
                            qkvb[2 * CHA + ei, pl.ds(i * 16, 16)] = (
                                qkvb[2 * CHA + ei, pl.ds(i * 16, 16)] * ev)
                    return 0

                lax.fori_loop(0, CHA // 16, grp, 0)
                pltpu.async_copy(qkvb.at[pl.ds(2 * CHA, CHA)],
                                 acc.at[ibd.at[0]], ssems[b], add=True)
            return 0

        lax.fori_loop(0, NCHA // 2, pair, 0)
        pltpu.make_async_copy(bufs[1].at[pl.ds(2 * CHA, CHA)],
                              acc.at[pl.ds(0, CHA)], ssems[1]).wait()
        plsc.subcore_barrier()
        pltpu.sync_copy(acc.at[pl.ds(s * 640, 640)],
                        agg_hbm.at[c, pl.ds(s * 640, 640)])
        pltpu.sync_copy(dn, den_hbm.at[w])

    return attn_pass


_make_sc_sconv = functools.cache(_make_sc_sconv)
_make_sc_attn = functools.cache(_make_sc_attn)


def _sc_sconv(y, gidx, didx):
    return _make_sc_sconv()(y, gidx, didx)


def _sc_attn(tqkv, sidx, didx):
    return _make_sc_attn()(tqkv, sidx, didx)


def kernel(x, edge_index, kernel_offsets, W1, b1, W2, b2, Wr1, br1, Wr2, br2,
           Wq, bq, Wk, bk, Wv, bv, Wo, bo):
    src = edge_index[0]
    dst = edge_index[1]
    npad_e = EPAD - E
    src_p = jnp.concatenate([src, jnp.zeros((npad_e,), jnp.int32)])
    koff_p = jnp.concatenate([kernel_offsets,
                              jnp.zeros((npad_e,), jnp.int32)])
    dst_p = jnp.concatenate([dst, jnp.full((npad_e,), N, jnp.int32)])
    gidx = (koff_p * NPAD + src_p).reshape(EPAD // CHS, CHS)
    didx_s = dst_p.reshape(EPAD // CHS, CHS)
    sidx = src_p.reshape(EPAD // CHS, CHS)

    Wcat1 = jnp.transpose(Wr1, (1, 0, 2)).reshape(H, K * H)
    Wcat2 = jnp.transpose(Wr2, (1, 0, 2)).reshape(H, K * H)

    inv_s = 1.0 / jnp.sqrt(jnp.float32(H))
    Wqkv = jnp.stack([Wq * inv_s, Wk, Wv])
    bqkv = jnp.stack([bq * inv_s, bk, bv]).reshape(3, 1, H)

    x_p = jnp.pad(x, ((0, NPAD - N), (0, 0)))
    h = _conv1(x_p, W1, b1)

    y1 = _taps_from_h(h, Wcat1).reshape(K * NPAD, H)
    p1 = _sc_sconv(y1, gidx, didx_s)
    y2 = _mid(p1, br1, Wcat2).reshape(K * NPAD, H)
    p2 = _sc_sconv(y2, gidx, didx_s)

    qkv = _qkv(h, Wqkv, bqkv)
    aggp, denp = _sc_attn(qkv.reshape(3 * NPAD, H), sidx, didx_s)

    return _final(x_p, h, p2, aggp, denp, br2, Wo, bo, W2, b2)[:N]
